# baseline (speedup 1.0000x reference)
import functools
import math

import jax
import jax.numpy as jnp
from jax import lax
from jax.experimental import pallas as pl
from jax.experimental.pallas import tpu as pltpu
from jax.experimental.pallas import tpu_sc as plsc

_C = 1.0
_VOCAB = 1000000
_DIM = 64
_BATCH = 4096
_SEQ = 50

_NC = 2
_NS = 16
_NW = _NC * _NS

_TOKENS = _BATCH * _SEQ
_PER_W = _TOKENS // _NW
_CHUNK = 128
_NCHUNK = _PER_W // _CHUNK
_NSLOT = 5

_BBLK = 4096
_NB = _BATCH // _BBLK


_RB = 2048
_DGRID = (_VOCAB + 2 * _RB - 1) // (2 * _RB)
_VROWS = _DGRID * _RB


def _detile_body(x1_ref, x2_ref, out_ref):
  t1 = jnp.transpose(x1_ref[...])
  t2 = jnp.transpose(x2_ref[...])
  out_ref[...] = jnp.concatenate([t1, t2], axis=1)


def _tc_detile(embT):
  return pl.pallas_call(
      _detile_body,
      grid=(_DGRID,),
      in_specs=[
          pl.BlockSpec((_DIM, _RB), lambda i: (0, 2 * i)),
          pl.BlockSpec((_DIM, _RB),
                       lambda i: (0, jnp.minimum(2 * i + 1, 2 * _DGRID - 2))),
      ],
      out_specs=pl.BlockSpec((_RB, 2 * _DIM), lambda i: (i, 0)),
      out_shape=jax.ShapeDtypeStruct((_VROWS, 2 * _DIM), jnp.float32),
  )(embT, embT)


def _sc_gather(idx3, table):
  mesh = plsc.VectorSubcoreMesh(
      core_axis_name="c", subcore_axis_name="s",
      num_cores=_NC, num_subcores=_NS)

  @functools.partial(
      pl.kernel,
      mesh=mesh,
      compiler_params=pltpu.CompilerParams(use_tc_tiling_on_sc=False),
      out_type=jax.ShapeDtypeStruct((_TOKENS, _DIM), jnp.float32),
      scratch_types=[
          pltpu.VMEM((_NCHUNK, _CHUNK), jnp.int32),
          pltpu.VMEM((_NSLOT, _CHUNK, _DIM), jnp.float32),
          pltpu.SemaphoreType.DMA((_NSLOT,)),
          pltpu.SemaphoreType.DMA((_NSLOT,)),
      ],
  )
  def k(idx_hbm, table_hbm, out_hbm, idx_v, rows_v, gsem, wsem):
    wid = lax.axis_index("s") * _NC + lax.axis_index("c")
    base = wid * _PER_W
    pltpu.sync_copy(idx_hbm.at[wid], idx_v)

    def gather(j, s):
      pltpu.make_async_copy(
          table_hbm.at[idx_v.at[j]], rows_v.at[s], gsem.at[s]).start()

    def writeback(j, s):
      return pltpu.make_async_copy(
          rows_v.at[s], out_hbm.at[pl.ds(base + j * _CHUNK, _CHUNK)],
          wsem.at[s])

    for s in range(_NSLOT):
      gather(s, s)

    def body(jj, _):
      for s in range(_NSLOT):
        j = jj * _NSLOT + s
        pltpu.make_async_copy(
            table_hbm.at[idx_v.at[j]], rows_v.at[s], gsem.at[s]).wait()
        writeback(j, s).start()

        @pl.when(jj < _NCHUNK // _NSLOT - 1)
        def _():
          writeback(j, s).wait()
          gather(j + _NSLOT, s)

      return 0

    lax.fori_loop(0, _NCHUNK // _NSLOT, body, 0)

    for s in range(_NSLOT):
      writeback(_NCHUNK - _NSLOT + s, s).wait()

  return k(idx3, table)


def _tc_body(x_ref, pe_ref, w_ref, b_ref, sc_ref, out_ref):
  x = x_ref[...]
  xt = jnp.transpose(x)
  pe = jnp.reshape(pe_ref[...], (_DIM, 1))
  y = xt + pe
  sq = y * y
  colsum = jnp.sum(sq, axis=0, keepdims=True)
  y0 = y[0:1, :]
  inv = lax.rsqrt(jnp.maximum(2.0 * y0 * y0 - colsum, 1e-7))
  yn = y * inv
  h = lax.dot_general(w_ref[...], yn, (((1,), (0,)), ((), ())),
                      preferred_element_type=jnp.float32) + b_ref[...]
  time = jax.nn.sigmoid(h[0:1, :]) * sc_ref[0, 0] + 1.1
  ssq = jnp.sum(h * h, axis=0, keepdims=True) - h[0:1, :] * h[0:1, :]
  s = (time * time - 1.0 / _C) / jnp.maximum(ssq, 1e-8)
  scaled = h * jnp.sqrt(s)
  rowid = lax.broadcasted_iota(jnp.int32, (_DIM, _BBLK), 0)
  outv = jnp.where(rowid == 0, time, scaled)
  out_ref[...] = jnp.reshape(outv[0:63, :], (1, 63, _BBLK))


def _tc_transform(gathered, pe_t, w_pad, b_col, scalars, interpret=False):
  return pl.pallas_call(
      _tc_body,
      grid=(_SEQ, _NB),
      in_specs=[
          pl.BlockSpec((_BBLK, _DIM), lambda i, j: (i * _NB + j, 0)),
          pl.BlockSpec((1, _DIM, 1), lambda i, j: (i, 0, 0)),
          pl.BlockSpec((_DIM, _DIM), lambda i, j: (0, 0)),
          pl.BlockSpec((_DIM, 1), lambda i, j: (0, 0)),
          pl.BlockSpec((1, 1), lambda i, j: (0, 0)),
      ],
      out_specs=pl.BlockSpec((1, 63, _BBLK), lambda i, j: (i, 0, j)),
      out_shape=jax.ShapeDtypeStruct((_SEQ, 63, _BATCH), jnp.float32),
      interpret=interpret,
  )(gathered, pe_t, w_pad, b_col, scalars)


def kernel(source, embedding, pos_enc, add_scale, W, b, point_scale):
  idx = jnp.transpose(source).reshape(-1).astype(jnp.int32)
  g = idx // (2 * _RB)
  k = idx % (2 * _RB)
  q = 2 * (g * _RB + k % _RB) + k // _RB
  idx3 = q.reshape(_NW, _NCHUNK, _CHUNK)

  table = _tc_detile(jnp.transpose(embedding)).reshape(2 * _VROWS, _DIM)
  gathered = _sc_gather(idx3, table)

  pe_t = (add_scale * pos_enc[:_SEQ, 0, :]).astype(jnp.float32)[:, :, None]
  w_pad = jnp.zeros((_DIM, _DIM), jnp.float32).at[:63, :].set(W)
  b_col = jnp.zeros((_DIM, 1), jnp.float32).at[:63, 0].set(b)
  scalars = jnp.exp(point_scale).reshape(1, 1)

  out = _tc_transform(gathered, pe_t, w_pad, b_col, scalars)
  return jnp.transpose(out, (2, 0, 1))


# speedup vs baseline: 1.3769x; 1.3769x over previous
import functools
import math

import jax
import jax.numpy as jnp
from jax import lax
from jax.experimental import pallas as pl
from jax.experimental.pallas import tpu as pltpu
from jax.experimental.pallas import tpu_sc as plsc

_C = 1.0
_VOCAB = 1000000
_DIM = 64
_BATCH = 4096
_SEQ = 50

_NC = 2
_NS = 16
_NW = _NC * _NS

_TOKENS = _BATCH * _SEQ
_PER_W = _TOKENS // _NW
_CHUNK = 128
_NCHUNK = _PER_W // _CHUNK
_NSLOT = 5

_BBLK = 4096
_NB = _BATCH // _BBLK


_RB = 2048
_DGRID = (_VOCAB + 2 * _RB - 1) // (2 * _RB)
_VROWS = _DGRID * _RB


def _detile_body(x_ref, out_ref):
  x = x_ref[...]
  x12 = jnp.concatenate([x[:, :_RB], x[:, _RB:]], axis=0)
  out_ref[...] = jnp.transpose(x12)


def _tc_detile(embT):
  return pl.pallas_call(
      _detile_body,
      grid=(_DGRID,),
      in_specs=[pl.BlockSpec((_DIM, 2 * _RB), lambda i: (0, i))],
      out_specs=pl.BlockSpec((_RB, 2 * _DIM), lambda i: (i, 0)),
      out_shape=jax.ShapeDtypeStruct((_VROWS, 2 * _DIM), jnp.float32),
  )(embT)


def _sc_gather(idx3, table):
  mesh = plsc.VectorSubcoreMesh(
      core_axis_name="c", subcore_axis_name="s",
      num_cores=_NC, num_subcores=_NS)

  @functools.partial(
      pl.kernel,
      mesh=mesh,
      compiler_params=pltpu.CompilerParams(use_tc_tiling_on_sc=False),
      out_type=jax.ShapeDtypeStruct((_TOKENS, _DIM), jnp.float32),
      scratch_types=[
          pltpu.VMEM((_NCHUNK, _CHUNK), jnp.int32),
          pltpu.VMEM((_NSLOT, _CHUNK, _DIM), jnp.float32),
          pltpu.SemaphoreType.DMA((_NSLOT,)),
          pltpu.SemaphoreType.DMA((_NSLOT,)),
      ],
  )
  def k(idx_hbm, table_hbm, out_hbm, idx_v, rows_v, gsem, wsem):
    wid = lax.axis_index("s") * _NC + lax.axis_index("c")
    base = wid * _PER_W
    pltpu.sync_copy(idx_hbm.at[wid], idx_v)

    def gather(j, s):
      pltpu.make_async_copy(
          table_hbm.at[idx_v.at[j]], rows_v.at[s], gsem.at[s]).start()

    def writeback(j, s):
      return pltpu.make_async_copy(
          rows_v.at[s], out_hbm.at[pl.ds(base + j * _CHUNK, _CHUNK)],
          wsem.at[s])

    for s in range(_NSLOT):
      gather(s, s)

    def body(jj, _):
      for s in range(_NSLOT):
        j = jj * _NSLOT + s
        pltpu.make_async_copy(
            table_hbm.at[idx_v.at[j]], rows_v.at[s], gsem.at[s]).wait()
        writeback(j, s).start()

        @pl.when(jj < _NCHUNK // _NSLOT - 1)
        def _():
          writeback(j, s).wait()
          gather(j + _NSLOT, s)

      return 0

    lax.fori_loop(0, _NCHUNK // _NSLOT, body, 0)

    for s in range(_NSLOT):
      writeback(_NCHUNK - _NSLOT + s, s).wait()

  return k(idx3, table)


_HB = _BATCH // 2


def _tc_body(x_ref, pe_ref, w_ref, b_ref, sc_ref, out_ref):
  x = x_ref[...]
  xt = jnp.transpose(x)
  pe = jnp.reshape(pe_ref[...], (_DIM, 1))
  pe2 = jnp.concatenate([pe, pe], axis=0)
  y = xt + pe2
  sq = y * y
  cs_e = jnp.sum(sq[0:_DIM, :], axis=0, keepdims=True)
  cs_o = jnp.sum(sq[_DIM:, :], axis=0, keepdims=True)
  y0e = y[0:1, :]
  y0o = y[_DIM:_DIM + 1, :]
  inv_e = lax.rsqrt(jnp.maximum(2.0 * y0e * y0e - cs_e, 1e-7))
  inv_o = lax.rsqrt(jnp.maximum(2.0 * y0o * y0o - cs_o, 1e-7))
  inv2 = jnp.concatenate([jnp.broadcast_to(inv_e, (_DIM, _HB)),
                          jnp.broadcast_to(inv_o, (_DIM, _HB))], axis=0)
  yn = y * inv2
  h = lax.dot_general(w_ref[...], yn, (((1,), (0,)), ((), ())),
                      preferred_element_type=jnp.float32) + b_ref[...]
  esc = sc_ref[0, 0]
  t_e = jax.nn.sigmoid(h[0:1, :]) * esc + 1.1
  t_o = jax.nn.sigmoid(h[_DIM:_DIM + 1, :]) * esc + 1.1
  ssq_e = (jnp.sum(h[0:_DIM, :] * h[0:_DIM, :], axis=0, keepdims=True)
           - h[0:1, :] * h[0:1, :])
  ssq_o = (jnp.sum(h[_DIM:, :] * h[_DIM:, :], axis=0, keepdims=True)
           - h[_DIM:_DIM + 1, :] * h[_DIM:_DIM + 1, :])
  s_e = (t_e * t_e - 1.0 / _C) / jnp.maximum(ssq_e, 1e-8)
  s_o = (t_o * t_o - 1.0 / _C) / jnp.maximum(ssq_o, 1e-8)
  rowid = lax.broadcasted_iota(jnp.int32, (_DIM, _HB), 0)
  out_e = jnp.where(rowid == 0, t_e, h[0:_DIM, :] * jnp.sqrt(s_e))
  out_o = jnp.where(rowid == 0, t_o, h[_DIM:, :] * jnp.sqrt(s_o))
  outv = jnp.concatenate([out_e[0:63, :], out_o[0:63, :]], axis=1)
  out_ref[...] = jnp.reshape(outv, (1, 63, _BATCH))


def _tc_transform(gathered2, pe_t, w_blk, b_blk, scalars, interpret=False):
  return pl.pallas_call(
      _tc_body,
      grid=(_SEQ,),
      in_specs=[
          pl.BlockSpec((_HB, 2 * _DIM), lambda i: (i, 0)),
          pl.BlockSpec((1, _DIM, 1), lambda i: (i, 0, 0)),
          pl.BlockSpec((2 * _DIM, 2 * _DIM), lambda i: (0, 0)),
          pl.BlockSpec((2 * _DIM, 1), lambda i: (0, 0)),
          pl.BlockSpec((1, 1), lambda i: (0, 0)),
      ],
      out_specs=pl.BlockSpec((1, 63, _BATCH), lambda i: (i, 0, 0)),
      out_shape=jax.ShapeDtypeStruct((_SEQ, 63, _BATCH), jnp.float32),
      interpret=interpret,
  )(gathered2, pe_t, w_blk, b_blk, scalars)


def kernel(source, embedding, pos_enc, add_scale, W, b, point_scale):
  sperm = jnp.stack([jnp.arange(_HB, dtype=jnp.int32),
                     _HB + jnp.arange(_HB, dtype=jnp.int32)], axis=1).reshape(-1)
  idx = jnp.transpose(source)[:, sperm].reshape(-1).astype(jnp.int32)
  g = idx // (2 * _RB)
  k = idx % (2 * _RB)
  q = 2 * (g * _RB + k % _RB) + k // _RB
  idx3 = q.reshape(_NW, _NCHUNK, _CHUNK)

  table = _tc_detile(jnp.transpose(embedding)).reshape(2 * _VROWS, _DIM)
  gathered = _sc_gather(idx3, table)
  gathered2 = gathered.reshape(_TOKENS // 2, 2 * _DIM)

  pe_t = (add_scale * pos_enc[:_SEQ, 0, :]).astype(jnp.float32)[:, :, None]
  w_pad = jnp.zeros((_DIM, _DIM), jnp.float32).at[:63, :].set(W)
  w_blk = jnp.zeros((2 * _DIM, 2 * _DIM), jnp.float32)
  w_blk = w_blk.at[:_DIM, :_DIM].set(w_pad).at[_DIM:, _DIM:].set(w_pad)
  b_col = jnp.zeros((_DIM, 1), jnp.float32).at[:63, 0].set(b)
  b_blk = jnp.concatenate([b_col, b_col], axis=0)
  scalars = jnp.exp(point_scale).reshape(1, 1)

  out = _tc_transform(gathered2, pe_t, w_blk, b_blk, scalars)
  return jnp.transpose(out, (2, 0, 1))


# speedup vs baseline: 1.6782x; 1.2188x over previous
import functools
import math

import jax
import jax.numpy as jnp
from jax import lax
from jax.experimental import pallas as pl
from jax.experimental.pallas import tpu as pltpu
from jax.experimental.pallas import tpu_sc as plsc

_C = 1.0
_VOCAB = 1000000
_DIM = 64
_BATCH = 4096
_SEQ = 50

_NC = 2
_NS = 16
_NW = _NC * _NS

_TOKENS = _BATCH * _SEQ
_PER_W = _TOKENS // _NW
_CHUNK = 128
_NCHUNK = _PER_W // _CHUNK
_NSLOT = 5

_BBLK = 4096
_NB = _BATCH // _BBLK


_RB = 4096
_DGRID = (_VOCAB + 2 * _RB - 1) // (2 * _RB)
_VROWS = _DGRID * _RB


def _detile_body(x_ref, out_ref):
  x = x_ref[...]
  x12 = jnp.concatenate([x[:, :_RB], x[:, _RB:]], axis=0)
  out_ref[...] = jnp.transpose(x12)


def _tc_detile(embT):
  return pl.pallas_call(
      _detile_body,
      grid=(_DGRID,),
      in_specs=[pl.BlockSpec((_DIM, 2 * _RB), lambda i: (0, i))],
      out_specs=pl.BlockSpec((_RB, 2 * _DIM), lambda i: (i, 0)),
      out_shape=jax.ShapeDtypeStruct((_VROWS, 2 * _DIM), jnp.float32),
  )(embT)


def _sc_gather(idx3, table):
  mesh = plsc.VectorSubcoreMesh(
      core_axis_name="c", subcore_axis_name="s",
      num_cores=_NC, num_subcores=_NS)

  @functools.partial(
      pl.kernel,
      mesh=mesh,
      compiler_params=pltpu.CompilerParams(use_tc_tiling_on_sc=False),
      out_type=jax.ShapeDtypeStruct((_TOKENS, _DIM), jnp.float32),
      scratch_types=[
          pltpu.VMEM((_NCHUNK, _CHUNK), jnp.int32),
          pltpu.VMEM((_NSLOT, _CHUNK, _DIM), jnp.float32),
          pltpu.SemaphoreType.DMA((_NSLOT,)),
          pltpu.SemaphoreType.DMA((_NSLOT,)),
      ],
  )
  def k(idx_hbm, table_hbm, out_hbm, idx_v, rows_v, gsem, wsem):
    wid = lax.axis_index("s") * _NC + lax.axis_index("c")
    base = wid * _PER_W
    pltpu.sync_copy(idx_hbm.at[wid], idx_v)

    def gather(j, s):
      pltpu.make_async_copy(
          table_hbm.at[idx_v.at[j]], rows_v.at[s], gsem.at[s]).start()

    def writeback(j, s):
      return pltpu.make_async_copy(
          rows_v.at[s], out_hbm.at[pl.ds(base + j * _CHUNK, _CHUNK)],
          wsem.at[s])

    for s in range(_NSLOT):
      gather(s, s)

    def body(jj, _):
      for s in range(_NSLOT):
        j = jj * _NSLOT + s
        pltpu.make_async_copy(
            table_hbm.at[idx_v.at[j]], rows_v.at[s], gsem.at[s]).wait()
        writeback(j, s).start()

        @pl.when(jj < _NCHUNK // _NSLOT - 1)
        def _():
          writeback(j, s).wait()
          gather(j + _NSLOT, s)

      return 0

    lax.fori_loop(0, _NCHUNK // _NSLOT, body, 0)

    for s in range(_NSLOT):
      writeback(_NCHUNK - _NSLOT + s, s).wait()

  return k(idx3, table)


_HB = _BATCH // 2


def _tc_body(x_ref, pe_ref, w_ref, b_ref, sc_ref, out_ref):
  x = x_ref[...]
  xt = jnp.transpose(x)
  pe = jnp.reshape(pe_ref[...], (_DIM, 1))
  pe2 = jnp.concatenate([pe, pe], axis=0)
  y = xt + pe2
  sq = y * y
  cs_e = jnp.sum(sq[0:_DIM, :], axis=0, keepdims=True)
  cs_o = jnp.sum(sq[_DIM:, :], axis=0, keepdims=True)
  y0e = y[0:1, :]
  y0o = y[_DIM:_DIM + 1, :]
  inv_e = lax.rsqrt(jnp.maximum(2.0 * y0e * y0e - cs_e, 1e-7))
  inv_o = lax.rsqrt(jnp.maximum(2.0 * y0o * y0o - cs_o, 1e-7))
  inv2 = jnp.concatenate([jnp.broadcast_to(inv_e, (_DIM, _HB)),
                          jnp.broadcast_to(inv_o, (_DIM, _HB))], axis=0)
  yn = y * inv2
  h = lax.dot_general(w_ref[...], yn, (((1,), (0,)), ((), ())),
                      preferred_element_type=jnp.float32) + b_ref[...]
  esc = sc_ref[0, 0]
  t_e = jax.nn.sigmoid(h[0:1, :]) * esc + 1.1
  t_o = jax.nn.sigmoid(h[_DIM:_DIM + 1, :]) * esc + 1.1
  ssq_e = (jnp.sum(h[0:_DIM, :] * h[0:_DIM, :], axis=0, keepdims=True)
           - h[0:1, :] * h[0:1, :])
  ssq_o = (jnp.sum(h[_DIM:, :] * h[_DIM:, :], axis=0, keepdims=True)
           - h[_DIM:_DIM + 1, :] * h[_DIM:_DIM + 1, :])
  s_e = (t_e * t_e - 1.0 / _C) / jnp.maximum(ssq_e, 1e-8)
  s_o = (t_o * t_o - 1.0 / _C) / jnp.maximum(ssq_o, 1e-8)
  rowid = lax.broadcasted_iota(jnp.int32, (_DIM, _HB), 0)
  out_e = jnp.where(rowid == 0, t_e, h[0:_DIM, :] * jnp.sqrt(s_e))
  out_o = jnp.where(rowid == 0, t_o, h[_DIM:, :] * jnp.sqrt(s_o))
  outv = jnp.concatenate([out_e[0:63, :], out_o[0:63, :]], axis=1)
  out_ref[...] = jnp.reshape(outv, (1, 63, _BATCH))


def _tc_transform(gathered2, pe_t, w_blk, b_blk, scalars, interpret=False):
  return pl.pallas_call(
      _tc_body,
      grid=(_SEQ,),
      in_specs=[
          pl.BlockSpec((_HB, 2 * _DIM), lambda i: (i, 0)),
          pl.BlockSpec((1, _DIM, 1), lambda i: (i, 0, 0)),
          pl.BlockSpec((2 * _DIM, 2 * _DIM), lambda i: (0, 0)),
          pl.BlockSpec((2 * _DIM, 1), lambda i: (0, 0)),
          pl.BlockSpec((1, 1), lambda i: (0, 0)),
      ],
      out_specs=pl.BlockSpec((1, 63, _BATCH), lambda i: (i, 0, 0)),
      out_shape=jax.ShapeDtypeStruct((_SEQ, 63, _BATCH), jnp.float32),
      interpret=interpret,
  )(gathered2, pe_t, w_blk, b_blk, scalars)


def kernel(source, embedding, pos_enc, add_scale, W, b, point_scale):
  sperm = jnp.stack([jnp.arange(_HB, dtype=jnp.int32),
                     _HB + jnp.arange(_HB, dtype=jnp.int32)], axis=1).reshape(-1)
  idx = jnp.transpose(source)[:, sperm].reshape(-1).astype(jnp.int32)
  g = idx // (2 * _RB)
  k = idx % (2 * _RB)
  q = 2 * (g * _RB + k % _RB) + k // _RB
  idx3 = q.reshape(_NW, _NCHUNK, _CHUNK)

  table = _tc_detile(jnp.transpose(embedding)).reshape(2 * _VROWS, _DIM)
  gathered = _sc_gather(idx3, table)
  gathered2 = gathered.reshape(_TOKENS // 2, 2 * _DIM)

  pe_t = (add_scale * pos_enc[:_SEQ, 0, :]).astype(jnp.float32)[:, :, None]
  w_pad = jnp.zeros((_DIM, _DIM), jnp.float32).at[:63, :].set(W)
  w_blk = jnp.zeros((2 * _DIM, 2 * _DIM), jnp.float32)
  w_blk = w_blk.at[:_DIM, :_DIM].set(w_pad).at[_DIM:, _DIM:].set(w_pad)
  b_col = jnp.zeros((_DIM, 1), jnp.float32).at[:63, 0].set(b)
  b_blk = jnp.concatenate([b_col, b_col], axis=0)
  scalars = jnp.exp(point_scale).reshape(1, 1)

  out = _tc_transform(gathered2, pe_t, w_blk, b_blk, scalars)
  return jnp.transpose(out, (2, 0, 1))


# speedup vs baseline: 1.8387x; 1.0956x over previous
import functools
import math

import jax
import jax.numpy as jnp
from jax import lax
from jax.experimental import pallas as pl
from jax.experimental.pallas import tpu as pltpu
from jax.experimental.pallas import tpu_sc as plsc

_C = 1.0
_VOCAB = 1000000
_DIM = 64
_BATCH = 4096
_SEQ = 50

_NC = 2
_NS = 16
_NW = _NC * _NS

_TOKENS = _BATCH * _SEQ
_PER_W = _TOKENS // _NW
_CHUNK = 128
_NCHUNK = _PER_W // _CHUNK
_NSLOT = 5

_BBLK = 4096
_NB = _BATCH // _BBLK


_RB = 8192
_DGRID = (_VOCAB + 2 * _RB - 1) // (2 * _RB)
_VROWS = _DGRID * _RB


def _detile_body(x_ref, out_ref):
  x = x_ref[...]
  x12 = jnp.concatenate([x[:, :_RB], x[:, _RB:]], axis=0)
  out_ref[...] = jnp.transpose(x12)


def _tc_detile(embT):
  return pl.pallas_call(
      _detile_body,
      grid=(_DGRID,),
      in_specs=[pl.BlockSpec((_DIM, 2 * _RB), lambda i: (0, i))],
      out_specs=pl.BlockSpec((_RB, 2 * _DIM), lambda i: (i, 0)),
      out_shape=jax.ShapeDtypeStruct((_VROWS, 2 * _DIM), jnp.float32),
  )(embT)


def _sc_gather(idx3, table):
  mesh = plsc.VectorSubcoreMesh(
      core_axis_name="c", subcore_axis_name="s",
      num_cores=_NC, num_subcores=_NS)

  @functools.partial(
      pl.kernel,
      mesh=mesh,
      compiler_params=pltpu.CompilerParams(use_tc_tiling_on_sc=False),
      out_type=jax.ShapeDtypeStruct((_TOKENS, _DIM), jnp.float32),
      scratch_types=[
          pltpu.VMEM((_NCHUNK, _CHUNK), jnp.int32),
          pltpu.VMEM((_NSLOT, _CHUNK, _DIM), jnp.float32),
          pltpu.SemaphoreType.DMA((_NSLOT,)),
          pltpu.SemaphoreType.DMA((_NSLOT,)),
      ],
  )
  def k(idx_hbm, table_hbm, out_hbm, idx_v, rows_v, gsem, wsem):
    wid = lax.axis_index("s") * _NC + lax.axis_index("c")
    base = wid * _PER_W
    pltpu.sync_copy(idx_hbm.at[wid], idx_v)

    def gather(j, s):
      pltpu.make_async_copy(
          table_hbm.at[idx_v.at[j]], rows_v.at[s], gsem.at[s]).start()

    def writeback(j, s):
      return pltpu.make_async_copy(
          rows_v.at[s], out_hbm.at[pl.ds(base + j * _CHUNK, _CHUNK)],
          wsem.at[s])

    for s in range(_NSLOT):
      gather(s, s)

    def body(jj, _):
      for s in range(_NSLOT):
        j = jj * _NSLOT + s
        pltpu.make_async_copy(
            table_hbm.at[idx_v.at[j]], rows_v.at[s], gsem.at[s]).wait()
        writeback(j, s).start()

        @pl.when(jj < _NCHUNK // _NSLOT - 1)
        def _():
          writeback(j, s).wait()
          gather(j + _NSLOT, s)

      return 0

    lax.fori_loop(0, _NCHUNK // _NSLOT, body, 0)

    for s in range(_NSLOT):
      writeback(_NCHUNK - _NSLOT + s, s).wait()

  return k(idx3, table)


_HB = _BATCH // 2


def _tc_body(x_ref, pe_ref, w_ref, b_ref, sc_ref, out_ref):
  x = x_ref[...]
  xt = jnp.transpose(x)
  pe = jnp.reshape(pe_ref[...], (_DIM, 1))
  pe2 = jnp.concatenate([pe, pe], axis=0)
  y = xt + pe2
  sq = y * y
  cs_e = jnp.sum(sq[0:_DIM, :], axis=0, keepdims=True)
  cs_o = jnp.sum(sq[_DIM:, :], axis=0, keepdims=True)
  y0e = y[0:1, :]
  y0o = y[_DIM:_DIM + 1, :]
  inv_e = lax.rsqrt(jnp.maximum(2.0 * y0e * y0e - cs_e, 1e-7))
  inv_o = lax.rsqrt(jnp.maximum(2.0 * y0o * y0o - cs_o, 1e-7))
  inv2 = jnp.concatenate([jnp.broadcast_to(inv_e, (_DIM, _HB)),
                          jnp.broadcast_to(inv_o, (_DIM, _HB))], axis=0)
  yn = y * inv2
  h = lax.dot_general(w_ref[...], yn, (((1,), (0,)), ((), ())),
                      preferred_element_type=jnp.float32) + b_ref[...]
  esc = sc_ref[0, 0]
  t_e = jax.nn.sigmoid(h[0:1, :]) * esc + 1.1
  t_o = jax.nn.sigmoid(h[_DIM:_DIM + 1, :]) * esc + 1.1
  ssq_e = (jnp.sum(h[0:_DIM, :] * h[0:_DIM, :], axis=0, keepdims=True)
           - h[0:1, :] * h[0:1, :])
  ssq_o = (jnp.sum(h[_DIM:, :] * h[_DIM:, :], axis=0, keepdims=True)
           - h[_DIM:_DIM + 1, :] * h[_DIM:_DIM + 1, :])
  s_e = (t_e * t_e - 1.0 / _C) / jnp.maximum(ssq_e, 1e-8)
  s_o = (t_o * t_o - 1.0 / _C) / jnp.maximum(ssq_o, 1e-8)
  rowid = lax.broadcasted_iota(jnp.int32, (_DIM, _HB), 0)
  out_e = jnp.where(rowid == 0, t_e, h[0:_DIM, :] * jnp.sqrt(s_e))
  out_o = jnp.where(rowid == 0, t_o, h[_DIM:, :] * jnp.sqrt(s_o))
  outv = jnp.concatenate([out_e[0:63, :], out_o[0:63, :]], axis=1)
  out_ref[...] = jnp.reshape(outv, (1, 63, _BATCH))


def _tc_transform(gathered2, pe_t, w_blk, b_blk, scalars, interpret=False):
  return pl.pallas_call(
      _tc_body,
      grid=(_SEQ,),
      in_specs=[
          pl.BlockSpec((_HB, 2 * _DIM), lambda i: (i, 0)),
          pl.BlockSpec((1, _DIM, 1), lambda i: (i, 0, 0)),
          pl.BlockSpec((2 * _DIM, 2 * _DIM), lambda i: (0, 0)),
          pl.BlockSpec((2 * _DIM, 1), lambda i: (0, 0)),
          pl.BlockSpec((1, 1), lambda i: (0, 0)),
      ],
      out_specs=pl.BlockSpec((1, 63, _BATCH), lambda i: (i, 0, 0)),
      out_shape=jax.ShapeDtypeStruct((_SEQ, 63, _BATCH), jnp.float32),
      interpret=interpret,
  )(gathered2, pe_t, w_blk, b_blk, scalars)


def kernel(source, embedding, pos_enc, add_scale, W, b, point_scale):
  sperm = jnp.stack([jnp.arange(_HB, dtype=jnp.int32),
                     _HB + jnp.arange(_HB, dtype=jnp.int32)], axis=1).reshape(-1)
  idx = jnp.transpose(source)[:, sperm].reshape(-1).astype(jnp.int32)
  g = idx // (2 * _RB)
  k = idx % (2 * _RB)
  q = 2 * (g * _RB + k % _RB) + k // _RB
  idx3 = q.reshape(_NW, _NCHUNK, _CHUNK)

  table = _tc_detile(jnp.transpose(embedding)).reshape(2 * _VROWS, _DIM)
  gathered = _sc_gather(idx3, table)
  gathered2 = gathered.reshape(_TOKENS // 2, 2 * _DIM)

  pe_t = (add_scale * pos_enc[:_SEQ, 0, :]).astype(jnp.float32)[:, :, None]
  w_pad = jnp.zeros((_DIM, _DIM), jnp.float32).at[:63, :].set(W)
  w_blk = jnp.zeros((2 * _DIM, 2 * _DIM), jnp.float32)
  w_blk = w_blk.at[:_DIM, :_DIM].set(w_pad).at[_DIM:, _DIM:].set(w_pad)
  b_col = jnp.zeros((_DIM, 1), jnp.float32).at[:63, 0].set(b)
  b_blk = jnp.concatenate([b_col, b_col], axis=0)
  scalars = jnp.exp(point_scale).reshape(1, 1)

  out = _tc_transform(gathered2, pe_t, w_blk, b_blk, scalars)
  return jnp.transpose(out, (2, 0, 1))


# speedup vs baseline: 1.8705x; 1.0173x over previous
import functools
import math

import jax
import jax.numpy as jnp
from jax import lax
from jax.experimental import pallas as pl
from jax.experimental.pallas import tpu as pltpu
from jax.experimental.pallas import tpu_sc as plsc

_C = 1.0
_VOCAB = 1000000
_DIM = 64
_BATCH = 4096
_SEQ = 50

_NC = 2
_NS = 16
_NW = _NC * _NS

_TOKENS = _BATCH * _SEQ
_PER_W = _TOKENS // _NW
_CHUNK = 128
_NCHUNK = _PER_W // _CHUNK
_NSLOT = 5

_BBLK = 4096
_NB = _BATCH // _BBLK


_RB = 16384
_DGRID = (_VOCAB + 2 * _RB - 1) // (2 * _RB)
_VROWS = _DGRID * _RB


def _detile_body(x_ref, out_ref):
  x = x_ref[...]
  x12 = jnp.concatenate([x[:, :_RB], x[:, _RB:]], axis=0)
  out_ref[...] = jnp.transpose(x12)


def _tc_detile(embT):
  return pl.pallas_call(
      _detile_body,
      grid=(_DGRID,),
      in_specs=[pl.BlockSpec((_DIM, 2 * _RB), lambda i: (0, i))],
      out_specs=pl.BlockSpec((_RB, 2 * _DIM), lambda i: (i, 0)),
      out_shape=jax.ShapeDtypeStruct((_VROWS, 2 * _DIM), jnp.float32),
  )(embT)


def _sc_gather(idx3, table):
  mesh = plsc.VectorSubcoreMesh(
      core_axis_name="c", subcore_axis_name="s",
      num_cores=_NC, num_subcores=_NS)

  @functools.partial(
      pl.kernel,
      mesh=mesh,
      compiler_params=pltpu.CompilerParams(use_tc_tiling_on_sc=False),
      out_type=jax.ShapeDtypeStruct((_TOKENS, _DIM), jnp.float32),
      scratch_types=[
          pltpu.VMEM((_NCHUNK, _CHUNK), jnp.int32),
          pltpu.VMEM((_NSLOT, _CHUNK, _DIM), jnp.float32),
          pltpu.SemaphoreType.DMA((_NSLOT,)),
          pltpu.SemaphoreType.DMA((_NSLOT,)),
      ],
  )
  def k(idx_hbm, table_hbm, out_hbm, idx_v, rows_v, gsem, wsem):
    wid = lax.axis_index("s") * _NC + lax.axis_index("c")
    base = wid * _PER_W
    pltpu.sync_copy(idx_hbm.at[wid], idx_v)

    def gather(j, s):
      pltpu.make_async_copy(
          table_hbm.at[idx_v.at[j]], rows_v.at[s], gsem.at[s]).start()

    def writeback(j, s):
      return pltpu.make_async_copy(
          rows_v.at[s], out_hbm.at[pl.ds(base + j * _CHUNK, _CHUNK)],
          wsem.at[s])

    for s in range(_NSLOT):
      gather(s, s)

    def body(jj, _):
      for s in range(_NSLOT):
        j = jj * _NSLOT + s
        pltpu.make_async_copy(
            table_hbm.at[idx_v.at[j]], rows_v.at[s], gsem.at[s]).wait()
        writeback(j, s).start()

        @pl.when(jj < _NCHUNK // _NSLOT - 1)
        def _():
          writeback(j, s).wait()
          gather(j + _NSLOT, s)

      return 0

    lax.fori_loop(0, _NCHUNK // _NSLOT, body, 0)

    for s in range(_NSLOT):
      writeback(_NCHUNK - _NSLOT + s, s).wait()

  return k(idx3, table)


_HB = _BATCH // 2


def _tc_body(x_ref, pe_ref, w_ref, b_ref, sc_ref, out_ref):
  x = x_ref[...]
  xt = jnp.transpose(x)
  pe = jnp.reshape(pe_ref[...], (_DIM, 1))
  pe2 = jnp.concatenate([pe, pe], axis=0)
  y = xt + pe2
  sq = y * y
  cs_e = jnp.sum(sq[0:_DIM, :], axis=0, keepdims=True)
  cs_o = jnp.sum(sq[_DIM:, :], axis=0, keepdims=True)
  y0e = y[0:1, :]
  y0o = y[_DIM:_DIM + 1, :]
  inv_e = lax.rsqrt(jnp.maximum(2.0 * y0e * y0e - cs_e, 1e-7))
  inv_o = lax.rsqrt(jnp.maximum(2.0 * y0o * y0o - cs_o, 1e-7))
  inv2 = jnp.concatenate([jnp.broadcast_to(inv_e, (_DIM, _HB)),
                          jnp.broadcast_to(inv_o, (_DIM, _HB))], axis=0)
  yn = y * inv2
  h = lax.dot_general(w_ref[...], yn, (((1,), (0,)), ((), ())),
                      preferred_element_type=jnp.float32) + b_ref[...]
  esc = sc_ref[0, 0]
  t_e = jax.nn.sigmoid(h[0:1, :]) * esc + 1.1
  t_o = jax.nn.sigmoid(h[_DIM:_DIM + 1, :]) * esc + 1.1
  ssq_e = (jnp.sum(h[0:_DIM, :] * h[0:_DIM, :], axis=0, keepdims=True)
           - h[0:1, :] * h[0:1, :])
  ssq_o = (jnp.sum(h[_DIM:, :] * h[_DIM:, :], axis=0, keepdims=True)
           - h[_DIM:_DIM + 1, :] * h[_DIM:_DIM + 1, :])
  s_e = (t_e * t_e - 1.0 / _C) / jnp.maximum(ssq_e, 1e-8)
  s_o = (t_o * t_o - 1.0 / _C) / jnp.maximum(ssq_o, 1e-8)
  rowid = lax.broadcasted_iota(jnp.int32, (_DIM, _HB), 0)
  out_e = jnp.where(rowid == 0, t_e, h[0:_DIM, :] * jnp.sqrt(s_e))
  out_o = jnp.where(rowid == 0, t_o, h[_DIM:, :] * jnp.sqrt(s_o))
  outv = jnp.concatenate([out_e[0:63, :], out_o[0:63, :]], axis=1)
  out_ref[...] = jnp.reshape(outv, (1, 63, _BATCH))


def _tc_transform(gathered2, pe_t, w_blk, b_blk, scalars, interpret=False):
  return pl.pallas_call(
      _tc_body,
      grid=(_SEQ,),
      in_specs=[
          pl.BlockSpec((_HB, 2 * _DIM), lambda i: (i, 0)),
          pl.BlockSpec((1, _DIM, 1), lambda i: (i, 0, 0)),
          pl.BlockSpec((2 * _DIM, 2 * _DIM), lambda i: (0, 0)),
          pl.BlockSpec((2 * _DIM, 1), lambda i: (0, 0)),
          pl.BlockSpec((1, 1), lambda i: (0, 0)),
      ],
      out_specs=pl.BlockSpec((1, 63, _BATCH), lambda i: (i, 0, 0)),
      out_shape=jax.ShapeDtypeStruct((_SEQ, 63, _BATCH), jnp.float32),
      interpret=interpret,
  )(gathered2, pe_t, w_blk, b_blk, scalars)


def kernel(source, embedding, pos_enc, add_scale, W, b, point_scale):
  sperm = jnp.stack([jnp.arange(_HB, dtype=jnp.int32),
                     _HB + jnp.arange(_HB, dtype=jnp.int32)], axis=1).reshape(-1)
  idx = jnp.transpose(source)[:, sperm].reshape(-1).astype(jnp.int32)
  g = idx // (2 * _RB)
  k = idx % (2 * _RB)
  q = 2 * (g * _RB + k % _RB) + k // _RB
  idx3 = q.reshape(_NW, _NCHUNK, _CHUNK)

  table = _tc_detile(jnp.transpose(embedding)).reshape(2 * _VROWS, _DIM)
  gathered = _sc_gather(idx3, table)
  gathered2 = gathered.reshape(_TOKENS // 2, 2 * _DIM)

  pe_t = (add_scale * pos_enc[:_SEQ, 0, :]).astype(jnp.float32)[:, :, None]
  w_pad = jnp.zeros((_DIM, _DIM), jnp.float32).at[:63, :].set(W)
  w_blk = jnp.zeros((2 * _DIM, 2 * _DIM), jnp.float32)
  w_blk = w_blk.at[:_DIM, :_DIM].set(w_pad).at[_DIM:, _DIM:].set(w_pad)
  b_col = jnp.zeros((_DIM, 1), jnp.float32).at[:63, 0].set(b)
  b_blk = jnp.concatenate([b_col, b_col], axis=0)
  scalars = jnp.exp(point_scale).reshape(1, 1)

  out = _tc_transform(gathered2, pe_t, w_blk, b_blk, scalars)
  return jnp.transpose(out, (2, 0, 1))


# speedup vs baseline: 1.8734x; 1.0015x over previous
import functools

import jax
import jax.numpy as jnp
from jax import lax
from jax.experimental import pallas as pl
from jax.experimental.pallas import tpu as pltpu
from jax.experimental.pallas import tpu_sc as plsc

_C = 1.0
_VOCAB = 1000000
_DIM = 64
_BATCH = 4096
_SEQ = 50

_NC = 2
_NS = 16
_NW = _NC * _NS

_TOKENS = _BATCH * _SEQ
_PER_W = _TOKENS // _NW
_CHUNK = 128
_NCHUNK = _PER_W // _CHUNK
_NSLOT = 5

_RB = 16384
_DGRID = (_VOCAB + 2 * _RB - 1) // (2 * _RB)
_VROWS = _DGRID * _RB


def _detile_body(x_ref, out_ref):
  x = x_ref[...]
  x12 = jnp.concatenate([x[:, :_RB], x[:, _RB:]], axis=0)
  out_ref[...] = jnp.transpose(x12)


def _tc_detile(embT):
  return pl.pallas_call(
      _detile_body,
      grid=(_DGRID,),
      in_specs=[pl.BlockSpec((_DIM, 2 * _RB), lambda i: (0, i))],
      out_specs=pl.BlockSpec((_RB, 2 * _DIM), lambda i: (i, 0)),
      out_shape=jax.ShapeDtypeStruct((_VROWS, 2 * _DIM), jnp.float32),
  )(embT)


def _sc_gather(idx3, table):
  mesh = plsc.VectorSubcoreMesh(
      core_axis_name="c", subcore_axis_name="s",
      num_cores=_NC, num_subcores=_NS)

  @functools.partial(
      pl.kernel,
      mesh=mesh,
      compiler_params=pltpu.CompilerParams(use_tc_tiling_on_sc=False),
      out_type=jax.ShapeDtypeStruct((_TOKENS, _DIM), jnp.float32),
      scratch_types=[
          pltpu.VMEM((_NCHUNK, _CHUNK), jnp.int32),
          pltpu.VMEM((_NSLOT, _CHUNK, _DIM), jnp.float32),
          pltpu.SemaphoreType.DMA((_NSLOT,)),
          pltpu.SemaphoreType.DMA((_NSLOT,)),
      ],
  )
  def k(idx_hbm, table_hbm, out_hbm, idx_v, rows_v, gsem, wsem):
    wid = lax.axis_index("s") * _NC + lax.axis_index("c")
    base = wid * _PER_W
    pltpu.sync_copy(idx_hbm.at[wid], idx_v)

    def gather(j, s):
      pltpu.make_async_copy(
          table_hbm.at[idx_v.at[j]], rows_v.at[s], gsem.at[s]).start()

    def writeback(j, s):
      return pltpu.make_async_copy(
          rows_v.at[s], out_hbm.at[pl.ds(base + j * _CHUNK, _CHUNK)],
          wsem.at[s])

    for s in range(_NSLOT):
      gather(s, s)

    def body(jj, _):
      for s in range(_NSLOT):
        j = jj * _NSLOT + s
        pltpu.make_async_copy(
            table_hbm.at[idx_v.at[j]], rows_v.at[s], gsem.at[s]).wait()
        writeback(j, s).start()

        @pl.when(jj < _NCHUNK // _NSLOT - 1)
        def _():
          writeback(j, s).wait()
          gather(j + _NSLOT, s)

      return 0

    lax.fori_loop(0, _NCHUNK // _NSLOT, body, 0)

    for s in range(_NSLOT):
      writeback(_NCHUNK - _NSLOT + s, s).wait()

  return k(idx3, table)


_HB = _BATCH // 2


def _tc_body(x_ref, pe_ref, w_ref, b_ref, sc_ref, out_ref):
  x = x_ref[...]
  xt = jnp.transpose(x)
  pe = jnp.reshape(pe_ref[...], (_DIM, 1))
  pe2 = jnp.concatenate([pe, pe], axis=0)
  y = xt + pe2
  sq = y * y
  cs_e = jnp.sum(sq[0:_DIM, :], axis=0, keepdims=True)
  cs_o = jnp.sum(sq[_DIM:, :], axis=0, keepdims=True)
  y0e = y[0:1, :]
  y0o = y[_DIM:_DIM + 1, :]
  inv_e = lax.rsqrt(jnp.maximum(2.0 * y0e * y0e - cs_e, 1e-7))
  inv_o = lax.rsqrt(jnp.maximum(2.0 * y0o * y0o - cs_o, 1e-7))
  inv2 = jnp.concatenate([jnp.broadcast_to(inv_e, (_DIM, _HB)),
                          jnp.broadcast_to(inv_o, (_DIM, _HB))], axis=0)
  yn = y * inv2
  h = lax.dot_general(w_ref[...], yn, (((1,), (0,)), ((), ())),
                      preferred_element_type=jnp.float32) + b_ref[...]
  esc = sc_ref[0, 0]
  t_e = jax.nn.sigmoid(h[0:1, :]) * esc + 1.1
  t_o = jax.nn.sigmoid(h[_DIM:_DIM + 1, :]) * esc + 1.1
  ssq_e = (jnp.sum(h[0:_DIM, :] * h[0:_DIM, :], axis=0, keepdims=True)
           - h[0:1, :] * h[0:1, :])
  ssq_o = (jnp.sum(h[_DIM:, :] * h[_DIM:, :], axis=0, keepdims=True)
           - h[_DIM:_DIM + 1, :] * h[_DIM:_DIM + 1, :])
  s_e = (t_e * t_e - 1.0 / _C) / jnp.maximum(ssq_e, 1e-8)
  s_o = (t_o * t_o - 1.0 / _C) / jnp.maximum(ssq_o, 1e-8)
  rowid = lax.broadcasted_iota(jnp.int32, (_DIM, _HB), 0)
  out_e = jnp.where(rowid == 0, t_e, h[0:_DIM, :] * jnp.sqrt(s_e))
  out_o = jnp.where(rowid == 0, t_o, h[_DIM:, :] * jnp.sqrt(s_o))
  outv = jnp.concatenate([out_e[0:63, :], out_o[0:63, :]], axis=1)
  out_ref[...] = jnp.reshape(outv, (1, 63, _BATCH))


def _tc_transform(gathered2, pe_t, w_blk, b_blk, scalars, interpret=False):
  return pl.pallas_call(
      _tc_body,
      grid=(_SEQ,),
      in_specs=[
          pl.BlockSpec((_HB, 2 * _DIM), lambda i: (i, 0)),
          pl.BlockSpec((1, _DIM, 1), lambda i: (i, 0, 0)),
          pl.BlockSpec((2 * _DIM, 2 * _DIM), lambda i: (0, 0)),
          pl.BlockSpec((2 * _DIM, 1), lambda i: (0, 0)),
          pl.BlockSpec((1, 1), lambda i: (0, 0)),
      ],
      out_specs=pl.BlockSpec((1, 63, _BATCH), lambda i: (i, 0, 0)),
      out_shape=jax.ShapeDtypeStruct((_SEQ, 63, _BATCH), jnp.float32),
      interpret=interpret,
  )(gathered2, pe_t, w_blk, b_blk, scalars)


def kernel(source, embedding, pos_enc, add_scale, W, b, point_scale):
  sperm = jnp.stack([jnp.arange(_HB, dtype=jnp.int32),
                     _HB + jnp.arange(_HB, dtype=jnp.int32)], axis=1).reshape(-1)
  idx = jnp.transpose(source)[:, sperm].reshape(-1).astype(jnp.int32)
  g = idx // (2 * _RB)
  k = idx % (2 * _RB)
  q = 2 * (g * _RB + k % _RB) + k // _RB
  idx3 = q.reshape(_NW, _NCHUNK, _CHUNK)

  table = _tc_detile(jnp.transpose(embedding)).reshape(2 * _VROWS, _DIM)
  gathered = _sc_gather(idx3, table)
  gathered2 = gathered.reshape(_TOKENS // 2, 2 * _DIM)

  pe_t = (add_scale * pos_enc[:_SEQ, 0, :]).astype(jnp.float32)[:, :, None]
  w_pad = jnp.zeros((_DIM, _DIM), jnp.float32).at[:63, :].set(W)
  w_blk = jnp.zeros((2 * _DIM, 2 * _DIM), jnp.float32)
  w_blk = w_blk.at[:_DIM, :_DIM].set(w_pad).at[_DIM:, _DIM:].set(w_pad)
  b_col = jnp.zeros((_DIM, 1), jnp.float32).at[:63, 0].set(b)
  b_blk = jnp.concatenate([b_col, b_col], axis=0)
  scalars = jnp.exp(point_scale).reshape(1, 1)

  out = _tc_transform(gathered2, pe_t, w_blk, b_blk, scalars)
  return jnp.transpose(out, (2, 0, 1))
